# baseline (speedup 1.0000x reference)
import functools

import jax
import jax.numpy as jnp
from jax import lax
from jax.experimental import pallas as pl
from jax.experimental.pallas import tpu as pltpu
from jax.experimental.pallas import tpu_sc as plsc

NC, NS, L = 2, 16, 16
NW = NC * NS
DH = 16
DHH = 8
DOUT = 2
BLK = 1024
BLK2 = 1280
DEG_CE = 2048
PROP_CE = 256
_SC_PARAMS = pltpu.CompilerParams(needs_layout_passes=False)


@functools.cache
def _deg_sc(n_pad: int, e_pad: int):
    nchunk = e_pad // NW // DEG_CE
    mesh = plsc.VectorSubcoreMesh(core_axis_name="c", subcore_axis_name="s")

    @functools.partial(
        pl.kernel,
        out_type=jax.ShapeDtypeStruct((NW * n_pad,), jnp.float32),
        mesh=mesh,
        compiler_params=_SC_PARAMS,
        scratch_types=[
            pltpu.VMEM((16, 128), jnp.int32),
            pltpu.VMEM((n_pad,), jnp.float32),
        ],
    )
    def deg_kernel(dst_hbm, out_hbm, dstb, hist):
        cid = lax.axis_index("c")
        sid = lax.axis_index("s")
        wid = cid * NS + sid
        ones = jnp.ones((16,), jnp.float32)

        @pl.loop(0, n_pad // 16, unroll=8)
        def _(i):
            hist[pl.ds(i * 16, 16)] = jnp.zeros((16,), jnp.float32)

        @pl.loop(0, nchunk)
        def _(jc):
            pltpu.sync_copy(
                dst_hbm.at[pl.ds(wid * (nchunk * 16) + jc * 16, 16)], dstb)

            @pl.loop(0, 16)
            def _(r):
                for c in range(8):
                    dv = dstb[r, pl.ds(c * 16, 16)]
                    plsc.addupdate_scatter(hist, [dv], ones)

        pltpu.sync_copy(hist, out_hbm.at[pl.ds(wid * n_pad, n_pad)])

    return deg_kernel


@functools.cache
def _prop_sc(n_pad: int, e_pad: int):
    nhalf8 = (n_pad // 2) * DHH
    ntab = n_pad * DHH
    nchunk = e_pad // 8 // PROP_CE
    mesh = plsc.VectorSubcoreMesh(core_axis_name="c", subcore_axis_name="s")

    @functools.partial(
        pl.kernel,
        out_type=jax.ShapeDtypeStruct((NW * nhalf8,), jnp.float32),
        mesh=mesh,
        compiler_params=_SC_PARAMS,
        scratch_types=[
            pltpu.VMEM((16, 128), jnp.int32),
            pltpu.VMEM((16, 128), jnp.int32),
            pltpu.VMEM((ntab,), jnp.float32),
            pltpu.VMEM((nhalf8,), jnp.float32),
        ],
    )
    def prop_kernel(g_hbm, sf_hbm, df_hbm, out_hbm, sfb, dfb, tab, acc):
        cid = lax.axis_index("c")
        sid = lax.axis_index("s")
        wid = cid * NS + sid
        h = sid // 8
        s = sid - h * 8
        lo8 = cid * nhalf8

        @pl.loop(0, nhalf8 // 16, unroll=8)
        def _(i):
            acc[pl.ds(i * 16, 16)] = jnp.zeros((16,), jnp.float32)

        pltpu.sync_copy(g_hbm.at[pl.ds(h * ntab, ntab)], tab)

        rows_per_shard = nchunk * 16

        @pl.loop(0, nchunk)
        def _(jc):
            base = s * rows_per_shard + jc * 16
            pltpu.sync_copy(sf_hbm.at[pl.ds(base, 16)], sfb)
            pltpu.sync_copy(df_hbm.at[pl.ds(base, 16)], dfb)

            @pl.loop(0, 16)
            def _(r):
                for c in range(8):
                    sv = sfb[r, pl.ds(c * 16, 16)]
                    dv = dfb[r, pl.ds(c * 16, 16)]
                    vals = plsc.load_gather(tab, [sv])
                    lv = dv - lo8
                    mask = (lv >= 0) & (lv < nhalf8)
                    lv = jnp.where(mask, lv, 0)
                    plsc.addupdate_scatter(acc, [lv], vals, mask=mask)

        pltpu.sync_copy(acc, out_hbm.at[pl.ds(wid * nhalf8, nhalf8)])

    return prop_kernel


def _tc1_body(x_ref, w1_ref, degp_ref, g1h_ref, dis_ref):
    deg = jnp.sum(degp_ref[...], axis=0, keepdims=True) + 1.0
    dis = (1.0 / jnp.sqrt(deg)).T
    h = jnp.dot(x_ref[...], w1_ref[...], preferred_element_type=jnp.float32, precision=lax.Precision.HIGHEST)
    g = h * dis
    g1h_ref[...] = jnp.stack([g[:, :DHH], g[:, DHH:]], axis=0)
    dis_ref[...] = dis


def _tc2_body(acca_ref, accb_ref, g1a_ref, g1b_ref, dis_ref, b1_ref,
              g2a_ref, g2b_ref):
    dis = dis_ref[...]
    acca = jnp.sum(acca_ref[0, 0], axis=0) + g1a_ref[0]
    ra = jnp.maximum(acca * dis + b1_ref[0:1, :DHH], 0.0)
    g2a_ref[...] = ra * dis
    accb = jnp.sum(accb_ref[0, 0], axis=0) + g1b_ref[0]
    rb = jnp.maximum(accb * dis + b1_ref[0:1, DHH:], 0.0)
    g2b_ref[...] = rb * dis


def _tc3_body(acca_ref, accb_ref, g2a_ref, g2b_ref, dis_ref,
              w2a_ref, w2b_ref, b2_ref, out_ref):
    pa = (jnp.sum(acca_ref[0, 0], axis=0) + g2a_ref[0]) * dis_ref[...]
    pb = (jnp.sum(accb_ref[0, 0], axis=0) + g2b_ref[0]) * dis_ref[...]
    z = (jnp.dot(pa, w2a_ref[...], preferred_element_type=jnp.float32, precision=lax.Precision.HIGHEST)
         + jnp.dot(pb, w2b_ref[...], preferred_element_type=jnp.float32, precision=lax.Precision.HIGHEST)
         + b2_ref[...])
    m = jnp.max(z, axis=1, keepdims=True)
    zs = z - m
    out_ref[...] = zs - jnp.log(jnp.sum(jnp.exp(zs), axis=1, keepdims=True))


def kernel(x, edge_index, W1, b1, W2, b2):
    n, din = x.shape
    e = edge_index.shape[1]
    n_pad = -(-(n + 1) // 2048) * 2048
    nhalf = n_pad // 2
    nhalf8 = nhalf * DHH
    e_pad = -(-e // (NW * DEG_CE)) * (NW * DEG_CE)

    src = edge_index[0].astype(jnp.int32)
    dst = edge_index[1].astype(jnp.int32)
    sp = jnp.concatenate([src, jnp.zeros((e_pad - e,), jnp.int32)])
    dp = jnp.concatenate([dst, jnp.full((e_pad - e,), n, jnp.int32)])
    dst_p = dp.reshape(e_pad // 128, 128)
    col = jnp.arange(DHH, dtype=jnp.int32)
    sf = (sp[:, None] * DHH + col).reshape(e_pad * DHH // 128, 128)
    df = (dp[:, None] * DHH + col).reshape(e_pad * DHH // 128, 128)

    degp = _deg_sc(n_pad, e_pad)(dst_p).reshape(NW, n_pad)

    grid1 = n_pad // BLK
    g1h, dis = pl.pallas_call(
        _tc1_body,
        grid=(grid1,),
        in_specs=[
            pl.BlockSpec((BLK, din), lambda i: (i, 0)),
            pl.BlockSpec((din, DH), lambda i: (0, 0)),
            pl.BlockSpec((NW, BLK), lambda i: (0, i)),
        ],
        out_specs=[
            pl.BlockSpec((2, BLK, DHH), lambda i: (0, i, 0)),
            pl.BlockSpec((BLK, 1), lambda i: (i, 0)),
        ],
        out_shape=[
            jax.ShapeDtypeStruct((2, n_pad, DHH), jnp.float32),
            jax.ShapeDtypeStruct((n_pad, 1), jnp.float32),
        ],
    )(x, W1, degp)

    prop = _prop_sc(n_pad, e_pad)
    accp1 = prop(g1h.reshape(-1), sf, df).reshape(NC, 2, 8, nhalf, DHH)

    nb = nhalf // BLK2
    grid2 = n_pad // BLK2
    g2a, g2b = pl.pallas_call(
        _tc2_body,
        grid=(grid2,),
        in_specs=[
            pl.BlockSpec((1, 1, 8, BLK2, DHH), lambda i: (i // nb, 0, 0, i % nb, 0)),
            pl.BlockSpec((1, 1, 8, BLK2, DHH), lambda i: (i // nb, 1, 0, i % nb, 0)),
            pl.BlockSpec((1, BLK2, DHH), lambda i: (0, i, 0)),
            pl.BlockSpec((1, BLK2, DHH), lambda i: (1, i, 0)),
            pl.BlockSpec((BLK2, 1), lambda i: (i, 0)),
            pl.BlockSpec((1, DH), lambda i: (0, 0)),
        ],
        out_specs=[
            pl.BlockSpec((BLK2, DHH), lambda i: (i, 0)),
            pl.BlockSpec((BLK2, DHH), lambda i: (i, 0)),
        ],
        out_shape=[
            jax.ShapeDtypeStruct((n_pad, DHH), jnp.float32),
            jax.ShapeDtypeStruct((n_pad, DHH), jnp.float32),
        ],
    )(accp1, accp1, g1h, g1h, dis, b1.reshape(1, DH))
    g2h = jnp.stack([g2a, g2b], axis=0)

    accp2 = prop(g2h.reshape(-1), sf, df).reshape(NC, 2, 8, nhalf, DHH)

    grid3 = -(-n // BLK2)
    nb = nhalf // BLK2
    out = pl.pallas_call(
        _tc3_body,
        grid=(grid3,),
        in_specs=[
            pl.BlockSpec((1, 1, 8, BLK2, DHH), lambda i: (i // nb, 0, 0, i % nb, 0)),
            pl.BlockSpec((1, 1, 8, BLK2, DHH), lambda i: (i // nb, 1, 0, i % nb, 0)),
            pl.BlockSpec((1, BLK2, DHH), lambda i: (0, i, 0)),
            pl.BlockSpec((1, BLK2, DHH), lambda i: (1, i, 0)),
            pl.BlockSpec((BLK2, 1), lambda i: (i, 0)),
            pl.BlockSpec((DHH, DOUT), lambda i: (0, 0)),
            pl.BlockSpec((DHH, DOUT), lambda i: (0, 0)),
            pl.BlockSpec((1, DOUT), lambda i: (0, 0)),
        ],
        out_specs=pl.BlockSpec((BLK2, DOUT), lambda i: (i, 0)),
        out_shape=jax.ShapeDtypeStruct((n, DOUT), jnp.float32),
    )(accp2, accp2, g2h, g2h, dis, W2[:DHH], W2[DHH:], b2.reshape(1, DOUT))

    return out


# speedup vs baseline: 1.5866x; 1.5866x over previous
import functools

import jax
import jax.numpy as jnp
from jax import lax
from jax.experimental import pallas as pl
from jax.experimental.pallas import tpu as pltpu
from jax.experimental.pallas import tpu_sc as plsc

NC, NS, L = 2, 16, 16
NW = NC * NS
DH = 16
DHH = 8
DOUT = 2
BLK = 1024
BLK2 = 1280
DEG_CE = 2048
PROP_CE = 256
_SC_PARAMS = pltpu.CompilerParams(needs_layout_passes=False)


@functools.cache
def _deg_sc(n_pad: int, e_pad: int):
    nchunk = e_pad // NW // DEG_CE
    mesh = plsc.VectorSubcoreMesh(core_axis_name="c", subcore_axis_name="s")

    @functools.partial(
        pl.kernel,
        out_type=jax.ShapeDtypeStruct((NW * n_pad,), jnp.float32),
        mesh=mesh,
        compiler_params=_SC_PARAMS,
        scratch_types=[
            pltpu.VMEM((16, 128), jnp.int32),
            pltpu.VMEM((n_pad,), jnp.float32),
        ],
    )
    def deg_kernel(dst_hbm, out_hbm, dstb, hist):
        cid = lax.axis_index("c")
        sid = lax.axis_index("s")
        wid = cid * NS + sid
        ones = jnp.ones((16,), jnp.float32)

        @pl.loop(0, n_pad // 16, unroll=8)
        def _(i):
            hist[pl.ds(i * 16, 16)] = jnp.zeros((16,), jnp.float32)

        @pl.loop(0, nchunk)
        def _(jc):
            pltpu.sync_copy(
                dst_hbm.at[pl.ds(wid * (nchunk * 16) + jc * 16, 16)], dstb)

            @pl.loop(0, 16)
            def _(r):
                for c in range(8):
                    dv = dstb[r, pl.ds(c * 16, 16)]
                    plsc.addupdate_scatter(hist, [dv], ones)

        pltpu.sync_copy(hist, out_hbm.at[pl.ds(wid * n_pad, n_pad)])

    return deg_kernel


@functools.cache
def _prop_sc(n: int, n_pad: int, e_pad: int):
    nhalf8 = (n_pad // 2) * DHH
    ntab = n_pad * DHH
    nstage = -(-(n * DHH) // 16) * 16
    nchunk = e_pad // 8 // PROP_CE
    mesh = plsc.VectorSubcoreMesh(core_axis_name="c", subcore_axis_name="s")

    @functools.partial(
        pl.kernel,
        out_type=jax.ShapeDtypeStruct((NW * nhalf8,), jnp.float32),
        mesh=mesh,
        compiler_params=_SC_PARAMS,
        scratch_types=[
            pltpu.VMEM((16, 128), jnp.int32),
            pltpu.VMEM((16, 128), jnp.int32),
            pltpu.VMEM((16, 128), jnp.int32),
            pltpu.VMEM((16, 128), jnp.int32),
            pltpu.VMEM((nstage,), jnp.float32),
            pltpu.VMEM((nhalf8,), jnp.float32),
            pltpu.SemaphoreType.DMA,
            pltpu.SemaphoreType.DMA,
        ],
    )
    def prop_kernel(g_hbm, sf_hbm, df_hbm, out_hbm,
                    sfb0, dfb0, sfb1, dfb1, tab, acc, semA, semB):
        cid = lax.axis_index("c")
        sid = lax.axis_index("s")
        wid = cid * NS + sid
        h = sid // 8
        s = sid - h * 8
        lo8 = cid * nhalf8

        @pl.loop(0, nhalf8 // 16, unroll=8)
        def _(i):
            acc[pl.ds(i * 16, 16)] = jnp.zeros((16,), jnp.float32)

        pltpu.sync_copy(g_hbm.at[pl.ds(h * ntab, nstage)], tab)

        rows_per_shard = nchunk * 16
        base0 = s * rows_per_shard

        def process(sfb, dfb):
            @pl.loop(0, 16)
            def _(r):
                for c in range(8):
                    sv = sfb[r, pl.ds(c * 16, 16)]
                    dv = dfb[r, pl.ds(c * 16, 16)]
                    vals = plsc.load_gather(tab, [sv])
                    lv = dv - lo8
                    mask = (lv >= 0) & (lv < nhalf8)
                    lv = jnp.where(mask, lv, 0)
                    plsc.addupdate_scatter(acc, [lv], vals, mask=mask)

        def start(base, sfb, dfb, sem):
            pltpu.async_copy(sf_hbm.at[pl.ds(base, 16)], sfb, sem)
            pltpu.async_copy(df_hbm.at[pl.ds(base, 16)], dfb, sem)

        def drain(sfb, dfb, sem):
            pltpu.make_async_copy(sf_hbm.at[pl.ds(0, 16)], sfb, sem).wait()
            pltpu.make_async_copy(df_hbm.at[pl.ds(0, 16)], dfb, sem).wait()

        start(base0, sfb0, dfb0, semA)

        @pl.loop(0, nchunk // 2)
        def _(jj):
            start(base0 + (2 * jj + 1) * 16, sfb1, dfb1, semB)
            drain(sfb0, dfb0, semA)
            process(sfb0, dfb0)

            @pl.when(jj < nchunk // 2 - 1)
            def _():
                start(base0 + (2 * jj + 2) * 16, sfb0, dfb0, semA)

            drain(sfb1, dfb1, semB)
            process(sfb1, dfb1)

        pltpu.sync_copy(acc, out_hbm.at[pl.ds(wid * nhalf8, nhalf8)])

    return prop_kernel


def _tc1_body(x_ref, w1_ref, degp_ref, g1h_ref, dis_ref):
    deg = jnp.sum(degp_ref[...], axis=0, keepdims=True) + 1.0
    dis = (1.0 / jnp.sqrt(deg)).T
    h = jnp.dot(x_ref[...], w1_ref[...], preferred_element_type=jnp.float32, precision=lax.Precision.HIGHEST)
    g = h * dis
    g1h_ref[...] = jnp.stack([g[:, :DHH], g[:, DHH:]], axis=0)
    dis_ref[...] = dis


def _tc2_body(acca_ref, accb_ref, g1a_ref, g1b_ref, dis_ref, b1_ref,
              g2a_ref, g2b_ref):
    dis = dis_ref[...]
    acca = jnp.sum(acca_ref[0, 0], axis=0) + g1a_ref[0]
    ra = jnp.maximum(acca * dis + b1_ref[0:1, :DHH], 0.0)
    g2a_ref[...] = ra * dis
    accb = jnp.sum(accb_ref[0, 0], axis=0) + g1b_ref[0]
    rb = jnp.maximum(accb * dis + b1_ref[0:1, DHH:], 0.0)
    g2b_ref[...] = rb * dis


def _tc3_body(acca_ref, accb_ref, g2a_ref, g2b_ref, dis_ref,
              w2a_ref, w2b_ref, b2_ref, out_ref):
    pa = (jnp.sum(acca_ref[0, 0], axis=0) + g2a_ref[0]) * dis_ref[...]
    pb = (jnp.sum(accb_ref[0, 0], axis=0) + g2b_ref[0]) * dis_ref[...]
    z = (jnp.dot(pa, w2a_ref[...], preferred_element_type=jnp.float32, precision=lax.Precision.HIGHEST)
         + jnp.dot(pb, w2b_ref[...], preferred_element_type=jnp.float32, precision=lax.Precision.HIGHEST)
         + b2_ref[...])
    m = jnp.max(z, axis=1, keepdims=True)
    zs = z - m
    out_ref[...] = zs - jnp.log(jnp.sum(jnp.exp(zs), axis=1, keepdims=True))


def kernel(x, edge_index, W1, b1, W2, b2):
    n, din = x.shape
    e = edge_index.shape[1]
    n_pad = -(-(n + 1) // 2048) * 2048
    nhalf = n_pad // 2
    nhalf8 = nhalf * DHH
    e_pad = -(-e // (NW * DEG_CE)) * (NW * DEG_CE)

    src = edge_index[0].astype(jnp.int32)
    dst = edge_index[1].astype(jnp.int32)
    sp = jnp.concatenate([src, jnp.zeros((e_pad - e,), jnp.int32)])
    dp = jnp.concatenate([dst, jnp.full((e_pad - e,), n, jnp.int32)])
    dst_p = dp.reshape(e_pad // 128, 128)
    col = jnp.arange(DHH, dtype=jnp.int32)
    sf = (sp[:, None] * DHH + col).reshape(e_pad * DHH // 128, 128)
    df = (dp[:, None] * DHH + col).reshape(e_pad * DHH // 128, 128)

    degp = _deg_sc(n_pad, e_pad)(dst_p).reshape(NW, n_pad)

    grid1 = n_pad // BLK
    g1h, dis = pl.pallas_call(
        _tc1_body,
        grid=(grid1,),
        in_specs=[
            pl.BlockSpec((BLK, din), lambda i: (i, 0)),
            pl.BlockSpec((din, DH), lambda i: (0, 0)),
            pl.BlockSpec((NW, BLK), lambda i: (0, i)),
        ],
        out_specs=[
            pl.BlockSpec((2, BLK, DHH), lambda i: (0, i, 0)),
            pl.BlockSpec((BLK, 1), lambda i: (i, 0)),
        ],
        out_shape=[
            jax.ShapeDtypeStruct((2, n_pad, DHH), jnp.float32),
            jax.ShapeDtypeStruct((n_pad, 1), jnp.float32),
        ],
    )(x, W1, degp)

    prop = _prop_sc(n, n_pad, e_pad)
    accp1 = prop(g1h.reshape(-1), sf, df).reshape(NC, 2, 8, nhalf, DHH)

    nb = nhalf // BLK2
    grid2 = n_pad // BLK2
    g2a, g2b = pl.pallas_call(
        _tc2_body,
        grid=(grid2,),
        in_specs=[
            pl.BlockSpec((1, 1, 8, BLK2, DHH), lambda i: (i // nb, 0, 0, i % nb, 0)),
            pl.BlockSpec((1, 1, 8, BLK2, DHH), lambda i: (i // nb, 1, 0, i % nb, 0)),
            pl.BlockSpec((1, BLK2, DHH), lambda i: (0, i, 0)),
            pl.BlockSpec((1, BLK2, DHH), lambda i: (1, i, 0)),
            pl.BlockSpec((BLK2, 1), lambda i: (i, 0)),
            pl.BlockSpec((1, DH), lambda i: (0, 0)),
        ],
        out_specs=[
            pl.BlockSpec((BLK2, DHH), lambda i: (i, 0)),
            pl.BlockSpec((BLK2, DHH), lambda i: (i, 0)),
        ],
        out_shape=[
            jax.ShapeDtypeStruct((n_pad, DHH), jnp.float32),
            jax.ShapeDtypeStruct((n_pad, DHH), jnp.float32),
        ],
    )(accp1, accp1, g1h, g1h, dis, b1.reshape(1, DH))
    g2h = jnp.stack([g2a, g2b], axis=0)

    accp2 = prop(g2h.reshape(-1), sf, df).reshape(NC, 2, 8, nhalf, DHH)

    grid3 = -(-n // BLK2)
    nb = nhalf // BLK2
    out = pl.pallas_call(
        _tc3_body,
        grid=(grid3,),
        in_specs=[
            pl.BlockSpec((1, 1, 8, BLK2, DHH), lambda i: (i // nb, 0, 0, i % nb, 0)),
            pl.BlockSpec((1, 1, 8, BLK2, DHH), lambda i: (i // nb, 1, 0, i % nb, 0)),
            pl.BlockSpec((1, BLK2, DHH), lambda i: (0, i, 0)),
            pl.BlockSpec((1, BLK2, DHH), lambda i: (1, i, 0)),
            pl.BlockSpec((BLK2, 1), lambda i: (i, 0)),
            pl.BlockSpec((DHH, DOUT), lambda i: (0, 0)),
            pl.BlockSpec((DHH, DOUT), lambda i: (0, 0)),
            pl.BlockSpec((1, DOUT), lambda i: (0, 0)),
        ],
        out_specs=pl.BlockSpec((BLK2, DOUT), lambda i: (i, 0)),
        out_shape=jax.ShapeDtypeStruct((n, DOUT), jnp.float32),
    )(accp2, accp2, g2h, g2h, dis, W2[:DHH], W2[DHH:], b2.reshape(1, DOUT))

    return out
